# Initial kernel scaffold; baseline (speedup 1.0000x reference)
import jax
import jax.numpy as jnp
from jax.experimental import pallas as pl


def kernel(x_0, x_1, x_2, x_node, edge_index_0, edge_index_1, edge_index_2, edge_index_12, edge_weight_0, edge_weight_1, edge_weight_2, W_s1s, b_s1s, W_s2s, b_s2s, W_s12s, b_s12s, att_vec):
    raise NotImplementedError("write your pallas kernel here")


# speedup vs baseline: 5.0501x; 5.0501x over previous
import functools

import jax
import jax.numpy as jnp
from jax import lax
from jax.experimental import pallas as pl
from jax.experimental.pallas import tpu as pltpu, tpu_sc as plsc

N_NODES = 10000
D = 128
E_EDGES = 320000
CH = 80
N_TILES = 16
EPT = E_EDGES // N_TILES
BLK = 2000
NBLK = EPT // BLK
CPB = BLK // CH
ROWS_PER_TILE = 640
FIN_CH = 80


def _sc_pass(table, si_h, di_h, w_h, xa_h, out_h,
             acc, cnt, si_v, di_v, w_v, di_row, rows_v, ones_v, facc, fx, fcnt, sem,
             *, weighted, has_xadd):
    t = lax.axis_index("s")

    zeros16 = jnp.zeros((16,), jnp.float32)

    def zrow_body(i, _):
        for k in range(D // 16):
            facc[i, pl.ds(k * 16, 16)] = zeros16
        return 0

    lax.fori_loop(0, FIN_CH, zrow_body, 0)
    for k in range(FIN_CH // 16):
        fcnt[pl.ds(k * 16, 16)] = zeros16
        ones_v[pl.ds(k * 16, 16)] = jnp.ones((16,), jnp.float32)

    base0 = t * ROWS_PER_TILE

    def init_body(c, _):
        b = base0 + c * FIN_CH

        @pl.when(b < N_NODES)
        def _():
            pltpu.sync_copy(facc, acc.at[pl.ds(b, FIN_CH), :])
            pltpu.sync_copy(fcnt, cnt.at[pl.ds(b, FIN_CH)])
        return 0

    lax.fori_loop(0, ROWS_PER_TILE // FIN_CH, init_body, 0)

    plsc.subcore_barrier()

    def blk_body(b, _):
        pltpu.sync_copy(si_h.at[t, pl.ds(b * BLK, BLK)], si_v)
        pltpu.sync_copy(di_h.at[t, pl.ds(b * BLK, BLK)], di_v)
        if weighted:
            pltpu.sync_copy(w_h.at[t, pl.ds(b * BLK, BLK)], w_v)

        def chunk_body(j, _):
            base = j * CH
            pltpu.async_copy(table.at[si_v.at[pl.ds(base, CH)]], rows_v, sem).wait()
            for k in range(CH // 16):
                di_row[0, pl.ds(k * 16, 16)] = di_v[pl.ds(base + k * 16, 16)]
            if weighted:
                def grp_body(g, _):
                    w16 = w_v[pl.ds(base + g * 16, 16)]
                    for l in range(16):
                        i = g * 16 + l
                        ws = w16[l]
                        for k in range(D // 16):
                            sl = pl.ds(k * 16, 16)
                            rows_v[i, sl] = rows_v[i, sl] * ws
                    return 0
                lax.fori_loop(0, CH // 16, grp_body, 0)
            pltpu.sync_copy(rows_v, acc.at[di_row.at[0]], add=True)
            pltpu.sync_copy(ones_v, cnt.at[di_row.at[0]], add=True)
            return 0

        lax.fori_loop(0, CPB, chunk_body, 0)
        return 0

    lax.fori_loop(0, NBLK, blk_body, 0)

    plsc.subcore_barrier()

    def fin_body(c, _):
        b = base0 + c * FIN_CH

        @pl.when(b < N_NODES)
        def _():
            pltpu.sync_copy(acc.at[pl.ds(b, FIN_CH), :], facc)
            pltpu.sync_copy(cnt.at[pl.ds(b, FIN_CH)], fcnt)
            if has_xadd:
                pltpu.sync_copy(xa_h.at[pl.ds(b, FIN_CH), :], fx)

            def grp_body(g, _):
                inv16 = 1.0 / jnp.maximum(fcnt[pl.ds(g * 16, 16)], 1.0)
                if has_xadd:
                    inv16 = inv16 * 0.5
                for l in range(16):
                    i = g * 16 + l
                    inv = inv16[l]
                    for k in range(D // 16):
                        sl = pl.ds(k * 16, 16)
                        v = facc[i, sl] * inv
                        if has_xadd:
                            v = v + fx[i, sl] * 0.5
                        facc[i, sl] = v
                return 0

            lax.fori_loop(0, FIN_CH // 16, grp_body, 0)
            pltpu.sync_copy(facc, out_h.at[pl.ds(b, FIN_CH), :])
        return 0

    lax.fori_loop(0, ROWS_PER_TILE // FIN_CH, fin_body, 0)


@functools.lru_cache(maxsize=None)
def _make_stage(w0, x0, w1, x1):
    mesh = plsc.VectorSubcoreMesh(core_axis_name="c", subcore_axis_name="s")
    f32 = jnp.float32
    nch = E_EDGES // CH

    @functools.partial(
        pl.kernel,
        out_type=(jax.ShapeDtypeStruct((N_NODES, D), f32),
                  jax.ShapeDtypeStruct((N_NODES, D), f32)),
        mesh=mesh,
        scratch_types=[
            pltpu.VMEM_SHARED((N_NODES, D), f32),
            pltpu.VMEM_SHARED((N_NODES,), f32),
            pltpu.VMEM((BLK,), jnp.int32),
            pltpu.VMEM((BLK,), jnp.int32),
            pltpu.VMEM((BLK,), f32),
            pltpu.VMEM((1, CH), jnp.int32),
            pltpu.VMEM((CH, D), f32),
            pltpu.VMEM((CH,), f32),
            pltpu.VMEM((FIN_CH, D), f32),
            pltpu.VMEM((FIN_CH, D), f32),
            pltpu.VMEM((FIN_CH,), f32),
            pltpu.SemaphoreType.DMA,
        ],
        compiler_params=pltpu.CompilerParams(use_tc_tiling_on_sc=False),
    )
    def stage(t0, si0, di0, wa0, xa0, t1, si1, di1, wa1, xa1,
              out0, out1,
              acc, cnt, si_v, di_v, w_v, di_row, rows_v, ones_v, facc, fx, fcnt, sem):
        core = lax.axis_index("c")
        scr = (acc, cnt, si_v, di_v, w_v, di_row, rows_v, ones_v, facc, fx, fcnt, sem)

        @pl.when(core == 0)
        def _():
            _sc_pass(t0, si0, di0, wa0, xa0, out0, *scr,
                     weighted=w0, has_xadd=x0)

        @pl.when(core == 1)
        def _():
            _sc_pass(t1, si1, di1, wa1, xa1, out1, *scr,
                     weighted=w1, has_xadd=x1)

    return stage


def _tc_epilogue(s1_ref, s2_ref, s12_ref, w_ref, b_ref, att_ref, out_ref):
    hs = []
    for k, pre in enumerate((s1_ref, s2_ref, s12_ref)):
        x = pre[...]
        w = w_ref[k]
        h = lax.dot_general(x, w, (((1,), (1,)), ((), ())),
                            preferred_element_type=jnp.float32)
        hs.append(jax.nn.relu(h + b_ref[k, :][None, :]))
    scores = [jnp.sum(h * att_ref[k, :][None, :], axis=1, keepdims=True)
              for k, h in enumerate(hs)]
    sc = jnp.concatenate(scores, axis=1)
    m = jnp.max(sc, axis=1, keepdims=True)
    e = jnp.exp(sc - m)
    wgt = e / jnp.sum(e, axis=1, keepdims=True)
    out_ref[...] = (hs[0] * wgt[:, 0:1] + hs[1] * wgt[:, 1:2]
                    + hs[2] * wgt[:, 2:3])


def kernel(x_0, x_1, x_2, x_node, edge_index_0, edge_index_1, edge_index_2,
           edge_index_12, edge_weight_0, edge_weight_1, edge_weight_2,
           W_s1s, b_s1s, W_s2s, b_s2s, W_s12s, b_s12s, att_vec):
    f32 = jnp.float32
    nch = E_EDGES // CH
    i32 = jnp.int32

    def r2(a):
        return a.astype(i32).reshape(N_TILES, EPT)

    e1s, e1d = r2(edge_index_1[0]), r2(edge_index_1[1])
    e2s, e2d = r2(edge_index_2[0]), r2(edge_index_2[1])
    e12s, e12d = r2(edge_index_12[0]), r2(edge_index_12[1])
    w1r = edge_weight_1.reshape(N_TILES, EPT)
    w2r = edge_weight_2.reshape(N_TILES, EPT)
    net1, net2 = _make_stage(True, True, True, True)(
        x_node, e1s, e1d, w1r, x_1,
        x_node, e2s, e2d, w2r, x_2)

    s1s_pre, net12 = _make_stage(False, False, False, True)(
        net1, e1d, e1s, w1r, x_1,
        net1, e12s, e12d, w2r, x_2)

    s2s_pre, s12s_pre = _make_stage(False, False, True, False)(
        net2, e2d, e2s, w1r, x_1,
        net12, e2d, e2s, w2r, x_2)

    Wstk = jnp.stack([W_s1s, W_s2s, W_s12s])
    bstk = jnp.zeros((8, D), f32).at[0].set(b_s1s).at[1].set(b_s2s).at[2].set(b_s12s)
    attp = jnp.zeros((8, D), f32).at[:3].set(att_vec)

    blk = 400
    grid = (N_NODES // blk,)
    h_final = pl.pallas_call(
        _tc_epilogue,
        grid=grid,
        in_specs=[
            pl.BlockSpec((blk, D), lambda i: (i, 0)),
            pl.BlockSpec((blk, D), lambda i: (i, 0)),
            pl.BlockSpec((blk, D), lambda i: (i, 0)),
            pl.BlockSpec((3, D, D), lambda i: (0, 0, 0)),
            pl.BlockSpec((8, D), lambda i: (0, 0)),
            pl.BlockSpec((8, D), lambda i: (0, 0)),
        ],
        out_specs=pl.BlockSpec((blk, D), lambda i: (i, 0)),
        out_shape=jax.ShapeDtypeStruct((N_NODES, D), f32),
    )(s1s_pre, s2s_pre, s12s_pre, Wstk, bstk, attp)
    return h_final


# speedup vs baseline: 6.6176x; 1.3104x over previous
import functools

import jax
import jax.numpy as jnp
from jax import lax
from jax.experimental import pallas as pl
from jax.experimental.pallas import tpu as pltpu, tpu_sc as plsc

N_NODES = 10000
D = 128
E_EDGES = 320000
CH = 80
N_TILES = 16
EPT = E_EDGES // N_TILES
BLK = 2000
NBLK = EPT // BLK
CPB = BLK // CH
ROWS_PER_TILE = 640
FIN_CH = 80


def _sc_pass(table, si_h, di_h, w_h, xa_h, out_h,
             acc, cnt, si0, si1, di0, di1, wv0, wv1, dr, rows0, rows1, ones_v,
             fcnt, gs0, gs1, ss0, ss1, osem, ls0, ls1,
             *, weighted, has_xadd):
    t = lax.axis_index("s")
    zeros16 = jnp.zeros((16,), jnp.float32)

    def zrow_body(i, _):
        for k in range(D // 16):
            rows0[i, pl.ds(k * 16, 16)] = zeros16
        return 0

    lax.fori_loop(0, FIN_CH, zrow_body, 0)
    for k in range(FIN_CH // 16):
        fcnt[pl.ds(k * 16, 16)] = zeros16
        ones_v[pl.ds(k * 16, 16)] = jnp.ones((16,), jnp.float32)

    base0 = t * ROWS_PER_TILE

    def init_body(c, _):
        b = base0 + c * FIN_CH

        @pl.when(b < N_NODES)
        def _():
            pltpu.sync_copy(rows0, acc.at[pl.ds(b, FIN_CH), :])
            pltpu.sync_copy(fcnt, cnt.at[pl.ds(b, FIN_CH)])
        return 0

    lax.fori_loop(0, ROWS_PER_TILE // FIN_CH, init_body, 0)

    plsc.subcore_barrier()

    def load_blk(b, sib, dib, wvb, lsem):
        pltpu.async_copy(si_h.at[t, pl.ds(b * BLK, BLK)], sib, lsem)
        pltpu.async_copy(di_h.at[t, pl.ds(b * BLK, BLK)], dib, lsem)
        if weighted:
            pltpu.async_copy(w_h.at[t, pl.ds(b * BLK, BLK)], wvb, lsem)

    def wait_blk(sib, dib, wvb, lsem):
        pltpu.make_async_copy(si_h.at[t, pl.ds(0, BLK)], sib, lsem).wait()
        pltpu.make_async_copy(di_h.at[t, pl.ds(0, BLK)], dib, lsem).wait()
        if weighted:
            pltpu.make_async_copy(w_h.at[t, pl.ds(0, BLK)], wvb, lsem).wait()

    def start_g(sib, c, rows, gsem):
        pltpu.async_copy(table.at[sib.at[pl.ds(c * CH, CH)]], rows, gsem)

    def wait_g(sib, rows, gsem):
        pltpu.make_async_copy(table.at[sib.at[pl.ds(0, CH)]], rows, gsem).wait()

    def prep(dib, wvb, c, p, rows):
        base = c * CH
        for k in range(CH // 16):
            dr[p, pl.ds(k * 16, 16)] = dib[pl.ds(base + k * 16, 16)]
        if weighted:
            def grp_body(g, _):
                w16 = wvb[pl.ds(base + g * 16, 16)]
                for l in range(16):
                    i = g * 16 + l
                    ws = w16[l]
                    for k in range(D // 16):
                        sl = pl.ds(k * 16, 16)
                        rows[i, sl] = rows[i, sl] * ws
                return 0
            lax.fori_loop(0, CH // 16, grp_body, 0)

    def start_s(rows, p, ssem):
        pltpu.async_copy(rows, acc.at[dr.at[p]], ssem, add=True)
        pltpu.async_copy(ones_v, cnt.at[dr.at[p]], osem, add=True)

    def wait_s(rows, p, ssem):
        pltpu.make_async_copy(rows, acc.at[dr.at[p]], ssem).wait()

    def run_blk(sib, dib, wvb):
        wait_g(sib, rows0, gs0)
        prep(dib, wvb, 0, 0, rows0)
        start_g(sib, 1, rows1, gs1)
        start_s(rows0, 0, ss0)

        def pair_body(j, _):
            cb = 2 * j + 1
            wait_g(sib, rows1, gs1)
            prep(dib, wvb, cb, 1, rows1)
            wait_s(rows0, 0, ss0)
            start_g(sib, cb + 1, rows0, gs0)
            start_s(rows1, 1, ss1)
            wait_g(sib, rows0, gs0)
            prep(dib, wvb, cb + 1, 0, rows0)
            wait_s(rows1, 1, ss1)

            @pl.when(cb + 2 < CPB)
            def _():
                start_g(sib, cb + 2, rows1, gs1)
            start_s(rows0, 0, ss0)
            return 0

        lax.fori_loop(0, (CPB - 1) // 2, pair_body, 0)
        wait_s(rows0, 0, ss0)

        def odrain(k, _):
            pltpu.make_async_copy(ones_v, cnt.at[dr.at[0]], osem).wait()
            return 0

        lax.fori_loop(0, CPB, odrain, 0)

    load_blk(0, si0, di0, wv0, ls0)

    def bp_body(bp, _):
        b0 = 2 * bp
        wait_blk(si0, di0, wv0, ls0)
        start_g(si0, 0, rows0, gs0)
        load_blk(b0 + 1, si1, di1, wv1, ls1)
        run_blk(si0, di0, wv0)

        wait_blk(si1, di1, wv1, ls1)
        start_g(si1, 0, rows0, gs0)

        @pl.when(bp < NBLK // 2 - 1)
        def _():
            load_blk(b0 + 2, si0, di0, wv0, ls0)
        run_blk(si1, di1, wv1)
        return 0

    lax.fori_loop(0, NBLK // 2, bp_body, 0)

    plsc.subcore_barrier()

    def fin_body(c, _):
        b = base0 + c * FIN_CH

        @pl.when(b < N_NODES)
        def _():
            pltpu.sync_copy(acc.at[pl.ds(b, FIN_CH), :], rows0)
            pltpu.sync_copy(cnt.at[pl.ds(b, FIN_CH)], fcnt)
            if has_xadd:
                pltpu.sync_copy(xa_h.at[pl.ds(b, FIN_CH), :], rows1)

            def grp_body(g, _):
                inv16 = 1.0 / jnp.maximum(fcnt[pl.ds(g * 16, 16)], 1.0)
                if has_xadd:
                    inv16 = inv16 * 0.5
                for l in range(16):
                    i = g * 16 + l
                    inv = inv16[l]
                    for k in range(D // 16):
                        sl = pl.ds(k * 16, 16)
                        v = rows0[i, sl] * inv
                        if has_xadd:
                            v = v + rows1[i, sl] * 0.5
                        rows0[i, sl] = v
                return 0

            lax.fori_loop(0, FIN_CH // 16, grp_body, 0)
            pltpu.sync_copy(rows0, out_h.at[pl.ds(b, FIN_CH), :])
        return 0

    lax.fori_loop(0, ROWS_PER_TILE // FIN_CH, fin_body, 0)


@functools.lru_cache(maxsize=None)
def _make_stage(w0, x0, w1, x1):
    mesh = plsc.VectorSubcoreMesh(core_axis_name="c", subcore_axis_name="s")
    f32 = jnp.float32
    nch = E_EDGES // CH

    @functools.partial(
        pl.kernel,
        out_type=(jax.ShapeDtypeStruct((N_NODES, D), f32),
                  jax.ShapeDtypeStruct((N_NODES, D), f32)),
        mesh=mesh,
        scratch_types=[
            pltpu.VMEM_SHARED((N_NODES, D), f32),
            pltpu.VMEM_SHARED((N_NODES,), f32),
            pltpu.VMEM((BLK,), jnp.int32),
            pltpu.VMEM((BLK,), jnp.int32),
            pltpu.VMEM((BLK,), jnp.int32),
            pltpu.VMEM((BLK,), jnp.int32),
            pltpu.VMEM((BLK,), f32),
            pltpu.VMEM((BLK,), f32),
            pltpu.VMEM((2, CH), jnp.int32),
            pltpu.VMEM((CH, D), f32),
            pltpu.VMEM((CH, D), f32),
            pltpu.VMEM((CH,), f32),
            pltpu.VMEM((FIN_CH,), f32),
            pltpu.SemaphoreType.DMA,
            pltpu.SemaphoreType.DMA,
            pltpu.SemaphoreType.DMA,
            pltpu.SemaphoreType.DMA,
            pltpu.SemaphoreType.DMA,
            pltpu.SemaphoreType.DMA,
            pltpu.SemaphoreType.DMA,
        ],
        compiler_params=pltpu.CompilerParams(use_tc_tiling_on_sc=False),
    )
    def stage(t0, si0_h, di0_h, wa0, xa0, t1, si1_h, di1_h, wa1, xa1,
              out0, out1,
              acc, cnt, si0, si1, di0, di1, wv0, wv1, dr, rows0, rows1,
              ones_v, fcnt, gs0, gs1, ss0, ss1, osem, ls0, ls1):
        core = lax.axis_index("c")
        scr = (acc, cnt, si0, si1, di0, di1, wv0, wv1, dr, rows0, rows1,
               ones_v, fcnt, gs0, gs1, ss0, ss1, osem, ls0, ls1)

        @pl.when(core == 0)
        def _():
            _sc_pass(t0, si0_h, di0_h, wa0, xa0, out0, *scr,
                     weighted=w0, has_xadd=x0)

        @pl.when(core == 1)
        def _():
            _sc_pass(t1, si1_h, di1_h, wa1, xa1, out1, *scr,
                     weighted=w1, has_xadd=x1)

    return stage


def _tc_epilogue(s1_ref, s2_ref, s12_ref, w_ref, b_ref, att_ref, out_ref):
    hs = []
    for k, pre in enumerate((s1_ref, s2_ref, s12_ref)):
        x = pre[...]
        w = w_ref[k]
        h = lax.dot_general(x, w, (((1,), (1,)), ((), ())),
                            preferred_element_type=jnp.float32)
        hs.append(jax.nn.relu(h + b_ref[k, :][None, :]))
    scores = [jnp.sum(h * att_ref[k, :][None, :], axis=1, keepdims=True)
              for k, h in enumerate(hs)]
    sc = jnp.concatenate(scores, axis=1)
    m = jnp.max(sc, axis=1, keepdims=True)
    e = jnp.exp(sc - m)
    wgt = e / jnp.sum(e, axis=1, keepdims=True)
    out_ref[...] = (hs[0] * wgt[:, 0:1] + hs[1] * wgt[:, 1:2]
                    + hs[2] * wgt[:, 2:3])


def kernel(x_0, x_1, x_2, x_node, edge_index_0, edge_index_1, edge_index_2,
           edge_index_12, edge_weight_0, edge_weight_1, edge_weight_2,
           W_s1s, b_s1s, W_s2s, b_s2s, W_s12s, b_s12s, att_vec):
    f32 = jnp.float32
    nch = E_EDGES // CH
    i32 = jnp.int32

    def r2(a):
        return a.astype(i32).reshape(N_TILES, EPT)

    e1s, e1d = r2(edge_index_1[0]), r2(edge_index_1[1])
    e2s, e2d = r2(edge_index_2[0]), r2(edge_index_2[1])
    e12s, e12d = r2(edge_index_12[0]), r2(edge_index_12[1])
    w1r = edge_weight_1.reshape(N_TILES, EPT)
    w2r = edge_weight_2.reshape(N_TILES, EPT)
    net1, net2 = _make_stage(True, True, True, True)(
        x_node, e1s, e1d, w1r, x_1,
        x_node, e2s, e2d, w2r, x_2)

    s1s_pre, net12 = _make_stage(False, False, False, True)(
        net1, e1d, e1s, w1r, x_1,
        net1, e12s, e12d, w2r, x_2)

    s2s_pre, s12s_pre = _make_stage(False, False, True, False)(
        net2, e2d, e2s, w1r, x_1,
        net12, e2d, e2s, w2r, x_2)

    Wstk = jnp.stack([W_s1s, W_s2s, W_s12s])
    bstk = jnp.zeros((8, D), f32).at[0].set(b_s1s).at[1].set(b_s2s).at[2].set(b_s12s)
    attp = jnp.zeros((8, D), f32).at[:3].set(att_vec)

    blk = 400
    grid = (N_NODES // blk,)
    h_final = pl.pallas_call(
        _tc_epilogue,
        grid=grid,
        in_specs=[
            pl.BlockSpec((blk, D), lambda i: (i, 0)),
            pl.BlockSpec((blk, D), lambda i: (i, 0)),
            pl.BlockSpec((blk, D), lambda i: (i, 0)),
            pl.BlockSpec((3, D, D), lambda i: (0, 0, 0)),
            pl.BlockSpec((8, D), lambda i: (0, 0)),
            pl.BlockSpec((8, D), lambda i: (0, 0)),
        ],
        out_specs=pl.BlockSpec((blk, D), lambda i: (i, 0)),
        out_shape=jax.ShapeDtypeStruct((N_NODES, D), f32),
    )(s1s_pre, s2s_pre, s12s_pre, Wstk, bstk, attp)
    return h_final


# speedup vs baseline: 8.8320x; 1.3346x over previous
import functools

import jax
import jax.numpy as jnp
from jax import lax
from jax.experimental import pallas as pl
from jax.experimental.pallas import tpu as pltpu, tpu_sc as plsc

N_NODES = 10000
D = 128
E_EDGES = 320000
CH = 80
N_TILES = 16
EPT = E_EDGES // N_TILES
BLK = 2000
NBLK = EPT // BLK
CPB = BLK // CH
ROWS_PER_TILE = 640
FIN_CH = 80


def _sc_pass(table, si_h, di_h, w_h, xa_h, out_h,
             acc, cnt, si0, si1, di0, di1, wv0, wv1, dr, rows0, rows1, rows2,
             ones_v, fcnt, gs0, gs1, gs2, ss0, ss1, ss2, osem, ls0, ls1,
             *, weighted, has_xadd):
    t = lax.axis_index("s")
    zeros16 = jnp.zeros((16,), jnp.float32)

    def zrow_body(i, _):
        for k in range(D // 16):
            rows0[i, pl.ds(k * 16, 16)] = zeros16
        return 0

    lax.fori_loop(0, FIN_CH, zrow_body, 0)
    for k in range(FIN_CH // 16):
        fcnt[pl.ds(k * 16, 16)] = zeros16
        ones_v[pl.ds(k * 16, 16)] = jnp.ones((16,), jnp.float32)

    base0 = t * ROWS_PER_TILE

    def init_body(c, _):
        b = base0 + c * FIN_CH

        @pl.when(b < N_NODES)
        def _():
            pltpu.sync_copy(rows0, acc.at[pl.ds(b, FIN_CH), :])
            pltpu.sync_copy(fcnt, cnt.at[pl.ds(b, FIN_CH)])
        return 0

    lax.fori_loop(0, ROWS_PER_TILE // FIN_CH, init_body, 0)

    plsc.subcore_barrier()

    def load_blk(b, sib, dib, wvb, lsem):
        pltpu.async_copy(si_h.at[t, pl.ds(b * BLK, BLK)], sib, lsem)
        pltpu.async_copy(di_h.at[t, pl.ds(b * BLK, BLK)], dib, lsem)
        if weighted:
            pltpu.async_copy(w_h.at[t, pl.ds(b * BLK, BLK)], wvb, lsem)

    def wait_blk(sib, dib, wvb, lsem):
        pltpu.make_async_copy(si_h.at[t, pl.ds(0, BLK)], sib, lsem).wait()
        pltpu.make_async_copy(di_h.at[t, pl.ds(0, BLK)], dib, lsem).wait()
        if weighted:
            pltpu.make_async_copy(w_h.at[t, pl.ds(0, BLK)], wvb, lsem).wait()

    def start_g(sib, c, rows, gsem):
        pltpu.async_copy(table.at[sib.at[pl.ds(c * CH, CH)]], rows, gsem)

    def wait_g(sib, rows, gsem):
        pltpu.make_async_copy(table.at[sib.at[pl.ds(0, CH)]], rows, gsem).wait()

    def prep(dib, wvb, c, p, rows):
        base = c * CH
        for k in range(CH // 16):
            dr[p, pl.ds(k * 16, 16)] = dib[pl.ds(base + k * 16, 16)]
        if weighted:
            def row_body(i, _):
                wsv = plsc.load_gather(
                    wvb, [jnp.full((16,), base + i, jnp.int32)])
                for k in range(D // 16):
                    sl = pl.ds(k * 16, 16)
                    rows[i, sl] = rows[i, sl] * wsv
                return 0
            lax.fori_loop(0, CH, row_body, 0, unroll=2)

    def start_s(rows, p, ssem):
        pltpu.async_copy(rows, acc.at[dr.at[p]], ssem, add=True)
        pltpu.async_copy(ones_v, cnt.at[dr.at[p]], osem, add=True)

    def wait_s(rows, p, ssem):
        pltpu.make_async_copy(rows, acc.at[dr.at[p]], ssem).wait()

    rows = (rows0, rows1, rows2)
    gs = (gs0, gs1, gs2)
    ss = (ss0, ss1, ss2)

    def run_blk(sib, dib, wvb):
        def chunk(c, p, first, last):
            pn = (p + 1) % 3
            if not first:
                wait_s(rows[pn], pn, ss[pn])
            if not last:
                start_g(sib, c + 1, rows[pn], gs[pn])
            wait_g(sib, rows[p], gs[p])
            prep(dib, wvb, c, p, rows[p])
            start_s(rows[p], p, ss[p])

        chunk(0, 0, True, False)
        chunk(1, 1, True, False)

        def trip_body(j, _):
            c = 3 * j + 2
            chunk(c, 2, False, False)
            chunk(c + 1, 0, False, False)
            chunk(c + 2, 1, False, False)
            return 0

        lax.fori_loop(0, (CPB - 4) // 3, trip_body, 0)
        chunk(CPB - 2, (CPB - 2) % 3, False, False)
        chunk(CPB - 1, (CPB - 1) % 3, False, True)
        wait_s(rows[(CPB - 2) % 3], (CPB - 2) % 3, ss[(CPB - 2) % 3])
        wait_s(rows[(CPB - 1) % 3], (CPB - 1) % 3, ss[(CPB - 1) % 3])

        def odrain(k, _):
            pltpu.make_async_copy(ones_v, cnt.at[dr.at[0]], osem).wait()
            return 0

        lax.fori_loop(0, CPB, odrain, 0)

    load_blk(0, si0, di0, wv0, ls0)

    def bp_body(bp, _):
        b0 = 2 * bp
        wait_blk(si0, di0, wv0, ls0)
        start_g(si0, 0, rows0, gs0)
        load_blk(b0 + 1, si1, di1, wv1, ls1)
        run_blk(si0, di0, wv0)

        wait_blk(si1, di1, wv1, ls1)
        start_g(si1, 0, rows0, gs0)

        @pl.when(bp < NBLK // 2 - 1)
        def _():
            load_blk(b0 + 2, si0, di0, wv0, ls0)
        run_blk(si1, di1, wv1)
        return 0

    lax.fori_loop(0, NBLK // 2, bp_body, 0)

    plsc.subcore_barrier()

    def fin_body(c, _):
        b = base0 + c * FIN_CH

        @pl.when(b < N_NODES)
        def _():
            pltpu.sync_copy(acc.at[pl.ds(b, FIN_CH), :], rows0)
            pltpu.sync_copy(cnt.at[pl.ds(b, FIN_CH)], fcnt)
            if has_xadd:
                pltpu.sync_copy(xa_h.at[pl.ds(b, FIN_CH), :], rows1)

            def frow_body(i, _):
                cv = plsc.load_gather(fcnt, [jnp.full((16,), i, jnp.int32)])
                inv = 1.0 / jnp.maximum(cv, 1.0)
                if has_xadd:
                    inv = inv * 0.5
                for k in range(D // 16):
                    sl = pl.ds(k * 16, 16)
                    v = rows0[i, sl] * inv
                    if has_xadd:
                        v = v + rows1[i, sl] * 0.5
                    rows0[i, sl] = v
                return 0

            lax.fori_loop(0, FIN_CH, frow_body, 0, unroll=2)
            pltpu.sync_copy(rows0, out_h.at[pl.ds(b, FIN_CH), :])
        return 0

    lax.fori_loop(0, ROWS_PER_TILE // FIN_CH, fin_body, 0)


@functools.lru_cache(maxsize=None)
def _make_stage(w0, x0, w1, x1):
    mesh = plsc.VectorSubcoreMesh(core_axis_name="c", subcore_axis_name="s")
    f32 = jnp.float32
    nch = E_EDGES // CH

    @functools.partial(
        pl.kernel,
        out_type=(jax.ShapeDtypeStruct((N_NODES, D), f32),
                  jax.ShapeDtypeStruct((N_NODES, D), f32)),
        mesh=mesh,
        scratch_types=[
            pltpu.VMEM_SHARED((N_NODES, D), f32),
            pltpu.VMEM_SHARED((N_NODES,), f32),
            pltpu.VMEM((BLK,), jnp.int32),
            pltpu.VMEM((BLK,), jnp.int32),
            pltpu.VMEM((BLK,), jnp.int32),
            pltpu.VMEM((BLK,), jnp.int32),
            pltpu.VMEM((BLK,), f32),
            pltpu.VMEM((BLK,), f32),
            pltpu.VMEM((3, CH), jnp.int32),
            pltpu.VMEM((CH, D), f32),
            pltpu.VMEM((CH, D), f32),
            pltpu.VMEM((CH, D), f32),
            pltpu.VMEM((CH,), f32),
            pltpu.VMEM((FIN_CH,), f32),
            pltpu.SemaphoreType.DMA,
            pltpu.SemaphoreType.DMA,
            pltpu.SemaphoreType.DMA,
            pltpu.SemaphoreType.DMA,
            pltpu.SemaphoreType.DMA,
            pltpu.SemaphoreType.DMA,
            pltpu.SemaphoreType.DMA,
            pltpu.SemaphoreType.DMA,
            pltpu.SemaphoreType.DMA,
        ],
        compiler_params=pltpu.CompilerParams(use_tc_tiling_on_sc=False,
                                             needs_layout_passes=False),
    )
    def stage(t0, si0_h, di0_h, wa0, xa0, t1, si1_h, di1_h, wa1, xa1,
              out0, out1,
              acc, cnt, si0, si1, di0, di1, wv0, wv1, dr, rows0, rows1, rows2,
              ones_v, fcnt, gs0, gs1, gs2, ss0, ss1, ss2, osem, ls0, ls1):
        core = lax.axis_index("c")
        scr = (acc, cnt, si0, si1, di0, di1, wv0, wv1, dr, rows0, rows1, rows2,
               ones_v, fcnt, gs0, gs1, gs2, ss0, ss1, ss2, osem, ls0, ls1)

        @pl.when(core == 0)
        def _():
            _sc_pass(t0, si0_h, di0_h, wa0, xa0, out0, *scr,
                     weighted=w0, has_xadd=x0)

        @pl.when(core == 1)
        def _():
            _sc_pass(t1, si1_h, di1_h, wa1, xa1, out1, *scr,
                     weighted=w1, has_xadd=x1)

    return stage


def _tc_epilogue(s1_ref, s2_ref, s12_ref, w_ref, b_ref, att_ref, out_ref):
    hs = []
    for k, pre in enumerate((s1_ref, s2_ref, s12_ref)):
        x = pre[...]
        w = w_ref[k]
        h = lax.dot_general(x, w, (((1,), (1,)), ((), ())),
                            preferred_element_type=jnp.float32)
        hs.append(jax.nn.relu(h + b_ref[k, :][None, :]))
    scores = [jnp.sum(h * att_ref[k, :][None, :], axis=1, keepdims=True)
              for k, h in enumerate(hs)]
    sc = jnp.concatenate(scores, axis=1)
    m = jnp.max(sc, axis=1, keepdims=True)
    e = jnp.exp(sc - m)
    wgt = e / jnp.sum(e, axis=1, keepdims=True)
    out_ref[...] = (hs[0] * wgt[:, 0:1] + hs[1] * wgt[:, 1:2]
                    + hs[2] * wgt[:, 2:3])


def kernel(x_0, x_1, x_2, x_node, edge_index_0, edge_index_1, edge_index_2,
           edge_index_12, edge_weight_0, edge_weight_1, edge_weight_2,
           W_s1s, b_s1s, W_s2s, b_s2s, W_s12s, b_s12s, att_vec):
    f32 = jnp.float32
    nch = E_EDGES // CH
    i32 = jnp.int32

    def r2(a):
        return a.astype(i32).reshape(N_TILES, EPT)

    e1s, e1d = r2(edge_index_1[0]), r2(edge_index_1[1])
    e2s, e2d = r2(edge_index_2[0]), r2(edge_index_2[1])
    e12s, e12d = r2(edge_index_12[0]), r2(edge_index_12[1])
    w1r = edge_weight_1.reshape(N_TILES, EPT)
    w2r = edge_weight_2.reshape(N_TILES, EPT)
    net1, net2 = _make_stage(True, True, True, True)(
        x_node, e1s, e1d, w1r, x_1,
        x_node, e2s, e2d, w2r, x_2)

    s1s_pre, net12 = _make_stage(False, False, False, True)(
        net1, e1d, e1s, w1r, x_1,
        net1, e12s, e12d, w2r, x_2)

    s2s_pre, s12s_pre = _make_stage(False, False, True, False)(
        net2, e2d, e2s, w1r, x_1,
        net12, e2d, e2s, w2r, x_2)

    Wstk = jnp.stack([W_s1s, W_s2s, W_s12s])
    bstk = jnp.zeros((8, D), f32).at[0].set(b_s1s).at[1].set(b_s2s).at[2].set(b_s12s)
    attp = jnp.zeros((8, D), f32).at[:3].set(att_vec)

    blk = 400
    grid = (N_NODES // blk,)
    h_final = pl.pallas_call(
        _tc_epilogue,
        grid=grid,
        in_specs=[
            pl.BlockSpec((blk, D), lambda i: (i, 0)),
            pl.BlockSpec((blk, D), lambda i: (i, 0)),
            pl.BlockSpec((blk, D), lambda i: (i, 0)),
            pl.BlockSpec((3, D, D), lambda i: (0, 0, 0)),
            pl.BlockSpec((8, D), lambda i: (0, 0)),
            pl.BlockSpec((8, D), lambda i: (0, 0)),
        ],
        out_specs=pl.BlockSpec((blk, D), lambda i: (i, 0)),
        out_shape=jax.ShapeDtypeStruct((N_NODES, D), f32),
    )(s1s_pre, s2s_pre, s12s_pre, Wstk, bstk, attp)
    return h_final
